# Initial kernel scaffold; baseline (speedup 1.0000x reference)
import jax
import jax.numpy as jnp
from jax.experimental import pallas as pl


def kernel(x, edge_index, Ws0, Wn0, b0, Ws1, Wn1, b1, Ws2, Wn2, b2, vW1, vb1, vW2, vb2, vW3, vb3):
    raise NotImplementedError("write your pallas kernel here")


# speedup vs baseline: 3.0377x; 3.0377x over previous
import jax
import jax.numpy as jnp
from jax import lax
from jax.experimental import pallas as pl
from jax.experimental.pallas import tpu as pltpu
from jax.experimental.pallas import tpu_sc as plsc

N = 10000
D = 128
H = 128
EMB = 64

NC = 2
NS = 16
NW = NC * NS

C = 128
E = 320000
K = 80
EPW = K * C
E_PAD = NW * EPW
DST_PAD = N

ACC_ROWS = 10112
RPT = ACC_ROWS // NS
_CHUNKS = [(0, 128), (128, 128), (256, 128), (384, 128), (512, 120)]

_f32 = jnp.float32


def _sc_aggregate(m, src3d, dst3d, zeros_blk, dim):
    mesh = plsc.VectorSubcoreMesh(core_axis_name="c", subcore_axis_name="s")

    def body(m_hbm, src_hbm, dst_hbm, z_hbm, acc_out,
             acc_sp, src_v, dst_v, rows_v, sem):
        cid = lax.axis_index("c")
        sid = lax.axis_index("s")
        wid = cid * NS + sid

        for off, sz in _CHUNKS:
            pltpu.sync_copy(z_hbm.at[pl.ds(0, sz)],
                            acc_sp.at[pl.ds(sid * RPT + off, sz)])

        pltpu.sync_copy(src_hbm.at[wid], src_v)
        pltpu.sync_copy(dst_hbm.at[wid], dst_v)

        plsc.subcore_barrier()

        def step(j, carry):
            pltpu.async_copy(m_hbm.at[src_v.at[j]], rows_v, sem).wait()
            pltpu.sync_copy(rows_v, acc_sp.at[dst_v.at[j]], add=True)
            return carry

        lax.fori_loop(0, K, step, 0)

        plsc.subcore_barrier()

        for off, sz in _CHUNKS:
            r0 = sid * RPT + off
            pltpu.sync_copy(acc_sp.at[pl.ds(r0, sz)], rows_v.at[pl.ds(0, sz)])
            pltpu.sync_copy(rows_v.at[pl.ds(0, sz)],
                            acc_out.at[cid, pl.ds(r0, sz)])

    fn = pl.kernel(
        body,
        out_type=jax.ShapeDtypeStruct((NC, ACC_ROWS, dim), _f32),
        mesh=mesh,
        scratch_types=[
            pltpu.VMEM_SHARED((ACC_ROWS, dim), _f32),
            pltpu.VMEM((K, C), jnp.int32),
            pltpu.VMEM((K, C), jnp.int32),
            pltpu.VMEM((C, dim), _f32),
            pltpu.SemaphoreType.DMA,
        ],
    )
    return fn(m, src3d, dst3d, zeros_blk)


def _sc_degree(dst3d, ones_blk, zeros_blk):
    mesh = plsc.VectorSubcoreMesh(core_axis_name="c", subcore_axis_name="s")

    def body(dst_hbm, ones_hbm, z_hbm, deg_out,
             deg_sp, dst_v, ones_v):
        cid = lax.axis_index("c")
        sid = lax.axis_index("s")
        wid = cid * NS + sid

        for off, sz in _CHUNKS:
            pltpu.sync_copy(z_hbm.at[pl.ds(0, sz)],
                            deg_sp.at[pl.ds(sid * RPT + off, sz)])
        pltpu.sync_copy(dst_hbm.at[wid], dst_v)
        pltpu.sync_copy(ones_hbm, ones_v)

        plsc.subcore_barrier()

        def step(j, carry):
            pltpu.sync_copy(ones_v, deg_sp.at[dst_v.at[j]], add=True)
            return carry

        lax.fori_loop(0, K, step, 0)

        plsc.subcore_barrier()

        for off, sz in _CHUNKS:
            r0 = sid * RPT + off
            pltpu.sync_copy(deg_sp.at[pl.ds(r0, sz)], ones_v.at[pl.ds(0, sz)])
            pltpu.sync_copy(ones_v.at[pl.ds(0, sz)],
                            deg_out.at[cid, pl.ds(r0, sz)])

    fn = pl.kernel(
        body,
        out_type=jax.ShapeDtypeStruct((NC, ACC_ROWS, H), _f32),
        mesh=mesh,
        scratch_types=[
            pltpu.VMEM_SHARED((ACC_ROWS, H), _f32),
            pltpu.VMEM((K, C), jnp.int32),
            pltpu.VMEM((C, H), _f32),
        ],
    )
    return fn(dst3d, ones_blk, zeros_blk)


def _dot(a, b):
    return jnp.dot(a.astype(jnp.bfloat16), b.astype(jnp.bfloat16),
                   preferred_element_type=_f32)


def _tc_layer(h, accp, degp, b, Ws, Wn, act):
    dout = Ws.shape[1]

    def body(h_ref, acc_ref, deg_ref, b_ref, ws_ref, wn_ref, o_ref):
        agg = acc_ref[0, :N, :] + acc_ref[1, :N, :]
        deg = deg_ref[0, :N, 0:1] + deg_ref[1, :N, 0:1]
        invd = 1.0 / jnp.maximum(deg, 1.0)
        out = (_dot(h_ref[...], ws_ref[...]) + _dot(agg * invd, wn_ref[...])
               + b_ref[...])
        o_ref[...] = jnp.maximum(out, 0.0) if act else out

    return pl.pallas_call(
        body,
        out_shape=jax.ShapeDtypeStruct((N, dout), _f32),
    )(h, accp, degp, b, Ws, Wn)


def _tc_final(h, accp, degp, b, Ws, Wn, vW1, vb1, vW2, vb2, vW3, vb3):
    def body(h_ref, acc_ref, deg_ref, b_ref, ws_ref, wn_ref,
             w1, bb1, w2, bb2, w3, bb3, o_ref):
        agg = acc_ref[0, :N, :] + acc_ref[1, :N, :]
        deg = deg_ref[0, :N, 0:1] + deg_ref[1, :N, 0:1]
        invd = 1.0 / jnp.maximum(deg, 1.0)
        h3 = (_dot(h_ref[...], ws_ref[...]) + _dot(agg * invd, wn_ref[...])
              + b_ref[...])
        g = jnp.mean(h3, axis=0, keepdims=True)
        v = jnp.maximum(_dot(g, w1[...]) + bb1[...], 0.0)
        v = jnp.maximum(_dot(v, w2[...]) + bb2[...], 0.0)
        o_ref[...] = _dot(v, w3[...]) + bb3[...]

    return pl.pallas_call(
        body,
        out_shape=jax.ShapeDtypeStruct((1, 1), _f32),
    )(h, accp, degp, b, Ws, Wn, vW1, vb1, vW2, vb2, vW3, vb3)


def kernel(x, edge_index, Ws0, Wn0, b0, Ws1, Wn1, b1, Ws2, Wn2, b2,
           vW1, vb1, vW2, vb2, vW3, vb3):
    src = edge_index[0]
    dst = edge_index[1]
    pad = E_PAD - E
    src3d = jnp.concatenate(
        [src, jnp.zeros((pad,), jnp.int32)]).reshape(NW, K, C)
    dst3d = jnp.concatenate(
        [dst, jnp.full((pad,), DST_PAD, jnp.int32)]).reshape(NW, K, C)

    z128 = jnp.zeros((C, H), _f32)
    ones128 = jnp.ones((C, H), _f32)

    degp = _sc_degree(dst3d, ones128, z128)

    accx = _sc_aggregate(x, src3d, dst3d, z128, dim=D)
    h1 = _tc_layer(x, accx, degp, b0.reshape(1, H), Ws0, Wn0, act=True)
    acc1 = _sc_aggregate(h1, src3d, dst3d, z128, dim=H)
    h2 = _tc_layer(h1, acc1, degp, b1.reshape(1, H), Ws1, Wn1, act=True)
    acc2 = _sc_aggregate(h2, src3d, dst3d, z128, dim=H)
    vh = vW1.shape[1]
    out = _tc_final(h2, acc2, degp, b2.reshape(1, EMB), Ws2, Wn2,
                    vW1, vb1.reshape(1, vh), vW2,
                    vb2.reshape(1, vh), vW3, vb3.reshape(1, 1))
    return out[0, 0]


# speedup vs baseline: 3.2420x; 1.0673x over previous
import jax
import jax.numpy as jnp
from jax import lax
from jax.experimental import pallas as pl
from jax.experimental.pallas import tpu as pltpu
from jax.experimental.pallas import tpu_sc as plsc

N = 10000
D = 128
H = 128
EMB = 64

NC = 2
NS = 16
NW = NC * NS

C = 128
E = 320000
K = 80
EPW = K * C
E_PAD = NW * EPW
DST_PAD = N

ACC_ROWS = 10112
RPT = ACC_ROWS // NS
_CHUNKS = [(0, 128), (128, 128), (256, 128), (384, 128), (512, 120)]

_f32 = jnp.float32


def _sc_aggregate(m, src3d, dst3d, zeros_blk, dim):
    mesh = plsc.VectorSubcoreMesh(core_axis_name="c", subcore_axis_name="s")
    KH = K // 2

    def body(m_hbm, src_hbm, dst_hbm, z_hbm, acc_out,
             acc_sp, src_v, dst_v, rows0, rows1, sem0, sem1):
        cid = lax.axis_index("c")
        sid = lax.axis_index("s")
        wid = cid * NS + sid

        for off, sz in _CHUNKS:
            pltpu.sync_copy(z_hbm.at[pl.ds(0, sz)],
                            acc_sp.at[pl.ds(sid * RPT + off, sz)])

        plsc.subcore_barrier()

        rows = (rows0, rows1)
        sems = (sem0, sem1)

        def gather(j, b):
            return pltpu.async_copy(m_hbm.at[src_v.at[j]], rows[b], sems[b])

        def scatter(j, b):
            pltpu.sync_copy(rows[b], acc_sp.at[dst_v.at[j]], add=True)

        for half in range(2):
            pltpu.sync_copy(src_hbm.at[wid, pl.ds(half * KH, KH)], src_v)
            pltpu.sync_copy(dst_hbm.at[wid, pl.ds(half * KH, KH)], dst_v)
            gather(0, 0)

            def step(i, carry):
                j0 = 2 * i
                pltpu.make_async_copy(m_hbm.at[pl.ds(0, C)], rows[0],
                                      sems[0]).wait()
                gather(j0 + 1, 1)
                scatter(j0, 0)
                pltpu.make_async_copy(m_hbm.at[pl.ds(0, C)], rows[1],
                                      sems[1]).wait()

                @pl.when(j0 + 2 < KH)
                def _():
                    gather(j0 + 2, 0)

                scatter(j0 + 1, 1)
                return carry

            lax.fori_loop(0, KH // 2, step, 0)

        plsc.subcore_barrier()

        for off, sz in _CHUNKS:
            r0 = sid * RPT + off
            pltpu.sync_copy(acc_sp.at[pl.ds(r0, sz)], rows0.at[pl.ds(0, sz)])
            pltpu.sync_copy(rows0.at[pl.ds(0, sz)],
                            acc_out.at[cid, pl.ds(r0, sz)])

    fn = pl.kernel(
        body,
        out_type=jax.ShapeDtypeStruct((NC, ACC_ROWS, dim), _f32),
        mesh=mesh,
        scratch_types=[
            pltpu.VMEM_SHARED((ACC_ROWS, dim), _f32),
            pltpu.VMEM((K // 2, C), jnp.int32),
            pltpu.VMEM((K // 2, C), jnp.int32),
            pltpu.VMEM((C, dim), _f32),
            pltpu.VMEM((C, dim), _f32),
            pltpu.SemaphoreType.DMA,
            pltpu.SemaphoreType.DMA,
        ],
    )
    return fn(m, src3d, dst3d, zeros_blk)


def _sc_degree(dst3d, ones_blk, zeros_blk):
    mesh = plsc.VectorSubcoreMesh(core_axis_name="c", subcore_axis_name="s")

    def body(dst_hbm, ones_hbm, z_hbm, deg_out,
             deg_sp, dst_v, ones_v):
        cid = lax.axis_index("c")
        sid = lax.axis_index("s")
        wid = cid * NS + sid

        for off, sz in _CHUNKS:
            pltpu.sync_copy(z_hbm.at[pl.ds(0, sz)],
                            deg_sp.at[pl.ds(sid * RPT + off, sz)])
        pltpu.sync_copy(dst_hbm.at[wid], dst_v)
        pltpu.sync_copy(ones_hbm, ones_v)

        plsc.subcore_barrier()

        def step(j, carry):
            pltpu.sync_copy(ones_v, deg_sp.at[dst_v.at[j]], add=True)
            return carry

        lax.fori_loop(0, K, step, 0)

        plsc.subcore_barrier()

        for off, sz in _CHUNKS:
            r0 = sid * RPT + off
            pltpu.sync_copy(deg_sp.at[pl.ds(r0, sz)], ones_v.at[pl.ds(0, sz)])
            pltpu.sync_copy(ones_v.at[pl.ds(0, sz)],
                            deg_out.at[cid, pl.ds(r0, sz)])

    fn = pl.kernel(
        body,
        out_type=jax.ShapeDtypeStruct((NC, ACC_ROWS, H), _f32),
        mesh=mesh,
        scratch_types=[
            pltpu.VMEM_SHARED((ACC_ROWS, H), _f32),
            pltpu.VMEM((K, C), jnp.int32),
            pltpu.VMEM((C, H), _f32),
        ],
    )
    return fn(dst3d, ones_blk, zeros_blk)


def _dot(a, b):
    return jnp.dot(a.astype(jnp.bfloat16), b.astype(jnp.bfloat16),
                   preferred_element_type=_f32)


def _tc_layer(h, accp, degp, b, Ws, Wn, act):
    dout = Ws.shape[1]

    def body(h_ref, acc_ref, deg_ref, b_ref, ws_ref, wn_ref, o_ref):
        agg = acc_ref[0, :N, :] + acc_ref[1, :N, :]
        deg = deg_ref[0, :N, 0:1] + deg_ref[1, :N, 0:1]
        invd = 1.0 / jnp.maximum(deg, 1.0)
        out = (_dot(h_ref[...], ws_ref[...]) + _dot(agg * invd, wn_ref[...])
               + b_ref[...])
        o_ref[...] = jnp.maximum(out, 0.0) if act else out

    return pl.pallas_call(
        body,
        out_shape=jax.ShapeDtypeStruct((N, dout), _f32),
    )(h, accp, degp, b, Ws, Wn)


def _tc_final(h, accp, degp, b, Ws, Wn, vW1, vb1, vW2, vb2, vW3, vb3):
    def body(h_ref, acc_ref, deg_ref, b_ref, ws_ref, wn_ref,
             w1, bb1, w2, bb2, w3, bb3, o_ref):
        agg = acc_ref[0, :N, :] + acc_ref[1, :N, :]
        deg = deg_ref[0, :N, 0:1] + deg_ref[1, :N, 0:1]
        invd = 1.0 / jnp.maximum(deg, 1.0)
        h3 = (_dot(h_ref[...], ws_ref[...]) + _dot(agg * invd, wn_ref[...])
              + b_ref[...])
        g = jnp.mean(h3, axis=0, keepdims=True)
        v = jnp.maximum(_dot(g, w1[...]) + bb1[...], 0.0)
        v = jnp.maximum(_dot(v, w2[...]) + bb2[...], 0.0)
        o_ref[...] = _dot(v, w3[...]) + bb3[...]

    return pl.pallas_call(
        body,
        out_shape=jax.ShapeDtypeStruct((1, 1), _f32),
    )(h, accp, degp, b, Ws, Wn, vW1, vb1, vW2, vb2, vW3, vb3)


def kernel(x, edge_index, Ws0, Wn0, b0, Ws1, Wn1, b1, Ws2, Wn2, b2,
           vW1, vb1, vW2, vb2, vW3, vb3):
    src = edge_index[0]
    dst = edge_index[1]
    pad = E_PAD - E
    src3d = jnp.concatenate(
        [src, jnp.zeros((pad,), jnp.int32)]).reshape(NW, K, C)
    dst3d = jnp.concatenate(
        [dst, jnp.full((pad,), DST_PAD, jnp.int32)]).reshape(NW, K, C)

    z128 = jnp.zeros((C, H), _f32)
    ones128 = jnp.ones((C, H), _f32)

    degp = _sc_degree(dst3d, ones128, z128)

    accx = _sc_aggregate(x, src3d, dst3d, z128, dim=D)
    h1 = _tc_layer(x, accx, degp, b0.reshape(1, H), Ws0, Wn0, act=True)
    acc1 = _sc_aggregate(h1, src3d, dst3d, z128, dim=H)
    h2 = _tc_layer(h1, acc1, degp, b1.reshape(1, H), Ws1, Wn1, act=True)
    acc2 = _sc_aggregate(h2, src3d, dst3d, z128, dim=H)
    vh = vW1.shape[1]
    out = _tc_final(h2, acc2, degp, b2.reshape(1, EMB), Ws2, Wn2,
                    vW1, vb1.reshape(1, vh), vW2,
                    vb2.reshape(1, vh), vW3, vb3.reshape(1, 1))
    return out[0, 0]


# speedup vs baseline: 3.3872x; 1.0448x over previous
import jax
import jax.numpy as jnp
from jax import lax
from jax.experimental import pallas as pl
from jax.experimental.pallas import tpu as pltpu
from jax.experimental.pallas import tpu_sc as plsc

N = 10000
D = 128
H = 128
EMB = 64

NC = 2
NS = 16
NW = NC * NS

C = 128
E = 320000
K = 80
EPW = K * C
E_PAD = NW * EPW
DST_PAD = N

ACC_ROWS = 10112
RPT = ACC_ROWS // NS
_CHUNKS = [(0, 128), (128, 128), (256, 128), (384, 128), (512, 120)]

_f32 = jnp.float32


def _sc_aggregate(m, src3d, dst3d, zeros_blk, dim):
    mesh = plsc.VectorSubcoreMesh(core_axis_name="c", subcore_axis_name="s")
    KH = K // 2

    def body(m_hbm, src_hbm, dst_hbm, z_hbm, acc_out,
             acc_sp, src_v, dst_v, rows0, rows1, sem0, sem1):
        cid = lax.axis_index("c")
        sid = lax.axis_index("s")
        wid = cid * NS + sid

        pltpu.sync_copy(z_hbm, rows0)
        for off, sz in _CHUNKS:
            pltpu.sync_copy(rows0.at[pl.ds(0, sz)],
                            acc_sp.at[pl.ds(sid * RPT + off, sz)])

        plsc.subcore_barrier()

        rows = (rows0, rows1)
        sems = (sem0, sem1)

        def gather(j, b):
            return pltpu.async_copy(m_hbm.at[src_v.at[j]], rows[b], sems[b])

        def wait(b):
            pltpu.make_async_copy(m_hbm.at[pl.ds(0, C)], rows[b],
                                  sems[b]).wait()

        def scatter(j, b):
            pltpu.sync_copy(rows[b], acc_sp.at[dst_v.at[j]], add=True)

        for half in range(2):
            pltpu.sync_copy(src_hbm.at[wid, pl.ds(half * KH, KH)], src_v)
            pltpu.sync_copy(dst_hbm.at[wid, pl.ds(half * KH, KH)], dst_v)
            gather(0, 0)
            gather(1, 1)

            def step(i, carry):
                j0 = 2 * i
                wait(0)
                scatter(j0, 0)

                @pl.when(j0 + 2 < KH)
                def _():
                    gather(j0 + 2, 0)

                wait(1)
                scatter(j0 + 1, 1)

                @pl.when(j0 + 3 < KH)
                def _():
                    gather(j0 + 3, 1)

                return carry

            lax.fori_loop(0, KH // 2, step, 0)

        plsc.subcore_barrier()

        for off, sz in _CHUNKS:
            r0 = sid * RPT + off
            pltpu.sync_copy(acc_sp.at[pl.ds(r0, sz)], rows0.at[pl.ds(0, sz)])
            pltpu.sync_copy(rows0.at[pl.ds(0, sz)],
                            acc_out.at[cid, pl.ds(r0, sz)])

    fn = pl.kernel(
        body,
        out_type=jax.ShapeDtypeStruct((NC, ACC_ROWS, dim), _f32),
        mesh=mesh,
        scratch_types=[
            pltpu.VMEM_SHARED((ACC_ROWS, dim), _f32),
            pltpu.VMEM((K // 2, C), jnp.int32),
            pltpu.VMEM((K // 2, C), jnp.int32),
            pltpu.VMEM((C, dim), _f32),
            pltpu.VMEM((C, dim), _f32),
            pltpu.SemaphoreType.DMA,
            pltpu.SemaphoreType.DMA,
        ],
    )
    return fn(m, src3d, dst3d, zeros_blk)


def _sc_degree(dst3d, ones_blk, zeros_blk):
    mesh = plsc.VectorSubcoreMesh(core_axis_name="c", subcore_axis_name="s")

    def body(dst_hbm, ones_hbm, z_hbm, deg_out,
             deg_sp, dst_v, ones_v):
        cid = lax.axis_index("c")
        sid = lax.axis_index("s")
        wid = cid * NS + sid

        for off, sz in _CHUNKS:
            pltpu.sync_copy(z_hbm.at[pl.ds(0, sz)],
                            deg_sp.at[pl.ds(sid * RPT + off, sz)])
        pltpu.sync_copy(dst_hbm.at[wid], dst_v)
        pltpu.sync_copy(ones_hbm, ones_v)

        plsc.subcore_barrier()

        def step(j, carry):
            pltpu.sync_copy(ones_v, deg_sp.at[dst_v.at[j]], add=True)
            return carry

        lax.fori_loop(0, K, step, 0)

        plsc.subcore_barrier()

        for off, sz in _CHUNKS:
            r0 = sid * RPT + off
            pltpu.sync_copy(deg_sp.at[pl.ds(r0, sz)], ones_v.at[pl.ds(0, sz)])
            pltpu.sync_copy(ones_v.at[pl.ds(0, sz)],
                            deg_out.at[cid, pl.ds(r0, sz)])

    fn = pl.kernel(
        body,
        out_type=jax.ShapeDtypeStruct((NC, ACC_ROWS, H), _f32),
        mesh=mesh,
        scratch_types=[
            pltpu.VMEM_SHARED((ACC_ROWS, H), _f32),
            pltpu.VMEM((K, C), jnp.int32),
            pltpu.VMEM((C, H), _f32),
        ],
    )
    return fn(dst3d, ones_blk, zeros_blk)


def _dot(a, b):
    return jnp.dot(a.astype(jnp.bfloat16), b.astype(jnp.bfloat16),
                   preferred_element_type=_f32)


def _tc_layer(h, accp, degp, b, Ws, Wn, act):
    dout = Ws.shape[1]

    def body(h_ref, acc_ref, deg_ref, b_ref, ws_ref, wn_ref, o_ref):
        agg = acc_ref[0, :N, :] + acc_ref[1, :N, :]
        deg = deg_ref[0, :N, 0:1] + deg_ref[1, :N, 0:1]
        invd = 1.0 / jnp.maximum(deg, 1.0)
        out = (_dot(h_ref[...], ws_ref[...]) + _dot(agg * invd, wn_ref[...])
               + b_ref[...])
        o_ref[...] = jnp.maximum(out, 0.0) if act else out

    return pl.pallas_call(
        body,
        out_shape=jax.ShapeDtypeStruct((N, dout), _f32),
    )(h, accp, degp, b, Ws, Wn)


def _tc_final(h, accp, degp, b, Ws, Wn, vW1, vb1, vW2, vb2, vW3, vb3):
    def body(h_ref, acc_ref, deg_ref, b_ref, ws_ref, wn_ref,
             w1, bb1, w2, bb2, w3, bb3, o_ref):
        agg = acc_ref[0, :N, :] + acc_ref[1, :N, :]
        deg = deg_ref[0, :N, 0:1] + deg_ref[1, :N, 0:1]
        invd = 1.0 / jnp.maximum(deg, 1.0)
        h3 = (_dot(h_ref[...], ws_ref[...]) + _dot(agg * invd, wn_ref[...])
              + b_ref[...])
        g = jnp.mean(h3, axis=0, keepdims=True)
        v = jnp.maximum(_dot(g, w1[...]) + bb1[...], 0.0)
        v = jnp.maximum(_dot(v, w2[...]) + bb2[...], 0.0)
        o_ref[...] = _dot(v, w3[...]) + bb3[...]

    return pl.pallas_call(
        body,
        out_shape=jax.ShapeDtypeStruct((1, 1), _f32),
    )(h, accp, degp, b, Ws, Wn, vW1, vb1, vW2, vb2, vW3, vb3)


def kernel(x, edge_index, Ws0, Wn0, b0, Ws1, Wn1, b1, Ws2, Wn2, b2,
           vW1, vb1, vW2, vb2, vW3, vb3):
    src = edge_index[0]
    dst = edge_index[1]
    pad = E_PAD - E
    src3d = jnp.concatenate(
        [src, jnp.zeros((pad,), jnp.int32)]).reshape(NW, K, C)
    dst3d = jnp.concatenate(
        [dst, jnp.full((pad,), DST_PAD, jnp.int32)]).reshape(NW, K, C)

    z128 = jnp.zeros((C, H), _f32)
    ones128 = jnp.ones((C, H), _f32)

    degp = _sc_degree(dst3d, ones128, z128)

    accx = _sc_aggregate(x, src3d, dst3d, z128, dim=D)
    h1 = _tc_layer(x, accx, degp, b0.reshape(1, H), Ws0, Wn0, act=True)
    acc1 = _sc_aggregate(h1, src3d, dst3d, z128, dim=H)
    h2 = _tc_layer(h1, acc1, degp, b1.reshape(1, H), Ws1, Wn1, act=True)
    acc2 = _sc_aggregate(h2, src3d, dst3d, z128, dim=H)
    vh = vW1.shape[1]
    out = _tc_final(h2, acc2, degp, b2.reshape(1, EMB), Ws2, Wn2,
                    vW1, vb1.reshape(1, vh), vW2,
                    vb2.reshape(1, vh), vW3, vb3.reshape(1, 1))
    return out[0, 0]


# speedup vs baseline: 3.3874x; 1.0001x over previous
import jax
import jax.numpy as jnp
from jax import lax
from jax.experimental import pallas as pl
from jax.experimental.pallas import tpu as pltpu
from jax.experimental.pallas import tpu_sc as plsc

N = 10000
D = 128
H = 128
EMB = 64

NC = 2
NS = 16
NW = NC * NS

C = 128
E = 320000
K = 80
EPW = K * C
E_PAD = NW * EPW
DST_PAD = N

ACC_ROWS = 10112
RPT = ACC_ROWS // NS
_CHUNKS = [(0, 128), (128, 128), (256, 128), (384, 128), (512, 120)]

_f32 = jnp.float32


def _sc_aggregate(m, src3d, dst3d, zeros_blk, dim):
    mesh = plsc.VectorSubcoreMesh(core_axis_name="c", subcore_axis_name="s")
    KH = K // 2

    def body(m_hbm, src_hbm, dst_hbm, z_hbm, acc_out,
             acc_sp, src_v, dst_v, rows0, rows1, sem0, sem1):
        cid = lax.axis_index("c")
        sid = lax.axis_index("s")
        wid = cid * NS + sid

        pltpu.sync_copy(z_hbm, rows0)
        for off, sz in _CHUNKS:
            pltpu.sync_copy(rows0.at[pl.ds(0, sz)],
                            acc_sp.at[pl.ds(sid * RPT + off, sz)])

        plsc.subcore_barrier()

        rows = (rows0, rows1)
        sems = (sem0, sem1)

        def gather(j, b):
            hc = C // 2
            pltpu.async_copy(m_hbm.at[src_v.at[j, pl.ds(0, hc)]],
                             rows[b].at[pl.ds(0, hc)], sems[b])
            pltpu.async_copy(m_hbm.at[src_v.at[j, pl.ds(hc, hc)]],
                             rows[b].at[pl.ds(hc, hc)], sems[b])

        def wait(b):
            pltpu.make_async_copy(m_hbm.at[pl.ds(0, C)], rows[b],
                                  sems[b]).wait()

        def scatter(j, b):
            pltpu.sync_copy(rows[b], acc_sp.at[dst_v.at[j]], add=True)

        for half in range(2):
            pltpu.sync_copy(src_hbm.at[wid, pl.ds(half * KH, KH)], src_v)
            pltpu.sync_copy(dst_hbm.at[wid, pl.ds(half * KH, KH)], dst_v)
            gather(0, 0)
            gather(1, 1)

            def step(i, carry):
                j0 = 2 * i
                wait(0)
                scatter(j0, 0)

                @pl.when(j0 + 2 < KH)
                def _():
                    gather(j0 + 2, 0)

                wait(1)
                scatter(j0 + 1, 1)

                @pl.when(j0 + 3 < KH)
                def _():
                    gather(j0 + 3, 1)

                return carry

            lax.fori_loop(0, KH // 2, step, 0)

        plsc.subcore_barrier()

        for off, sz in _CHUNKS:
            r0 = sid * RPT + off
            pltpu.sync_copy(acc_sp.at[pl.ds(r0, sz)], rows0.at[pl.ds(0, sz)])
            pltpu.sync_copy(rows0.at[pl.ds(0, sz)],
                            acc_out.at[cid, pl.ds(r0, sz)])

    fn = pl.kernel(
        body,
        out_type=jax.ShapeDtypeStruct((NC, ACC_ROWS, dim), _f32),
        mesh=mesh,
        scratch_types=[
            pltpu.VMEM_SHARED((ACC_ROWS, dim), _f32),
            pltpu.VMEM((K // 2, C), jnp.int32),
            pltpu.VMEM((K // 2, C), jnp.int32),
            pltpu.VMEM((C, dim), _f32),
            pltpu.VMEM((C, dim), _f32),
            pltpu.SemaphoreType.DMA,
            pltpu.SemaphoreType.DMA,
        ],
    )
    return fn(m, src3d, dst3d, zeros_blk)


def _sc_degree(dst3d, ones_blk, zeros_blk):
    mesh = plsc.VectorSubcoreMesh(core_axis_name="c", subcore_axis_name="s")

    def body(dst_hbm, ones_hbm, z_hbm, deg_out,
             deg_sp, dst_v, ones_v):
        cid = lax.axis_index("c")
        sid = lax.axis_index("s")
        wid = cid * NS + sid

        for off, sz in _CHUNKS:
            pltpu.sync_copy(z_hbm.at[pl.ds(0, sz)],
                            deg_sp.at[pl.ds(sid * RPT + off, sz)])
        pltpu.sync_copy(dst_hbm.at[wid], dst_v)
        pltpu.sync_copy(ones_hbm, ones_v)

        plsc.subcore_barrier()

        def step(j, carry):
            pltpu.sync_copy(ones_v, deg_sp.at[dst_v.at[j]], add=True)
            return carry

        lax.fori_loop(0, K, step, 0)

        plsc.subcore_barrier()

        for off, sz in _CHUNKS:
            r0 = sid * RPT + off
            pltpu.sync_copy(deg_sp.at[pl.ds(r0, sz)], ones_v.at[pl.ds(0, sz)])
            pltpu.sync_copy(ones_v.at[pl.ds(0, sz)],
                            deg_out.at[cid, pl.ds(r0, sz)])

    fn = pl.kernel(
        body,
        out_type=jax.ShapeDtypeStruct((NC, ACC_ROWS, H), _f32),
        mesh=mesh,
        scratch_types=[
            pltpu.VMEM_SHARED((ACC_ROWS, H), _f32),
            pltpu.VMEM((K, C), jnp.int32),
            pltpu.VMEM((C, H), _f32),
        ],
    )
    return fn(dst3d, ones_blk, zeros_blk)


def _dot(a, b):
    return jnp.dot(a.astype(jnp.bfloat16), b.astype(jnp.bfloat16),
                   preferred_element_type=_f32)


def _tc_layer(h, accp, degp, b, Ws, Wn, act):
    dout = Ws.shape[1]

    def body(h_ref, acc_ref, deg_ref, b_ref, ws_ref, wn_ref, o_ref):
        agg = acc_ref[0, :N, :] + acc_ref[1, :N, :]
        deg = deg_ref[0, :N, 0:1] + deg_ref[1, :N, 0:1]
        invd = 1.0 / jnp.maximum(deg, 1.0)
        out = (_dot(h_ref[...], ws_ref[...]) + _dot(agg * invd, wn_ref[...])
               + b_ref[...])
        o_ref[...] = jnp.maximum(out, 0.0) if act else out

    return pl.pallas_call(
        body,
        out_shape=jax.ShapeDtypeStruct((N, dout), _f32),
    )(h, accp, degp, b, Ws, Wn)


def _tc_final(h, accp, degp, b, Ws, Wn, vW1, vb1, vW2, vb2, vW3, vb3):
    def body(h_ref, acc_ref, deg_ref, b_ref, ws_ref, wn_ref,
             w1, bb1, w2, bb2, w3, bb3, o_ref):
        agg = acc_ref[0, :N, :] + acc_ref[1, :N, :]
        deg = deg_ref[0, :N, 0:1] + deg_ref[1, :N, 0:1]
        invd = 1.0 / jnp.maximum(deg, 1.0)
        h3 = (_dot(h_ref[...], ws_ref[...]) + _dot(agg * invd, wn_ref[...])
              + b_ref[...])
        g = jnp.mean(h3, axis=0, keepdims=True)
        v = jnp.maximum(_dot(g, w1[...]) + bb1[...], 0.0)
        v = jnp.maximum(_dot(v, w2[...]) + bb2[...], 0.0)
        o_ref[...] = _dot(v, w3[...]) + bb3[...]

    return pl.pallas_call(
        body,
        out_shape=jax.ShapeDtypeStruct((1, 1), _f32),
    )(h, accp, degp, b, Ws, Wn, vW1, vb1, vW2, vb2, vW3, vb3)


def kernel(x, edge_index, Ws0, Wn0, b0, Ws1, Wn1, b1, Ws2, Wn2, b2,
           vW1, vb1, vW2, vb2, vW3, vb3):
    src = edge_index[0]
    dst = edge_index[1]
    pad = E_PAD - E
    src3d = jnp.concatenate(
        [src, jnp.zeros((pad,), jnp.int32)]).reshape(NW, K, C)
    dst3d = jnp.concatenate(
        [dst, jnp.full((pad,), DST_PAD, jnp.int32)]).reshape(NW, K, C)

    z128 = jnp.zeros((C, H), _f32)
    ones128 = jnp.ones((C, H), _f32)

    degp = _sc_degree(dst3d, ones128, z128)

    accx = _sc_aggregate(x, src3d, dst3d, z128, dim=D)
    h1 = _tc_layer(x, accx, degp, b0.reshape(1, H), Ws0, Wn0, act=True)
    acc1 = _sc_aggregate(h1, src3d, dst3d, z128, dim=H)
    h2 = _tc_layer(h1, acc1, degp, b1.reshape(1, H), Ws1, Wn1, act=True)
    acc2 = _sc_aggregate(h2, src3d, dst3d, z128, dim=H)
    vh = vW1.shape[1]
    out = _tc_final(h2, acc2, degp, b2.reshape(1, EMB), Ws2, Wn2,
                    vW1, vb1.reshape(1, vh), vW2,
                    vb2.reshape(1, vh), vW3, vb3.reshape(1, 1))
    return out[0, 0]
